# baseline (speedup 1.0000x reference)
import functools

import jax
import jax.numpy as jnp
from jax import lax
from jax.experimental import pallas as pl
from jax.experimental.pallas import tpu as pltpu
from jax.experimental.pallas import tpu_sc as plsc

M, N, W, ITERS = 16, 64, 8, 10
NC, NS = 2, 16
NW = NC * NS
LANES = 16

B_SC = 1024
BPW = B_SC // NW
VREGS = BPW // LANES
TBLK = 1024


def _loo(vals, op):
    m01, m23 = op(vals[0], vals[1]), op(vals[2], vals[3])
    m45, m67 = op(vals[4], vals[5]), op(vals[6], vals[7])
    q03, q47 = op(m01, m23), op(m45, m67)
    h01, h23 = op(m23, q47), op(m01, q47)
    h45, h67 = op(q03, m67), op(q03, m45)
    return [
        op(vals[1], h01), op(vals[0], h01),
        op(vals[3], h23), op(vals[2], h23),
        op(vals[5], h45), op(vals[4], h45),
        op(vals[7], h67), op(vals[6], h67),
    ]


def _minsum_group(t, sval):
    av = [jnp.abs(tc) for tc in t]
    ng = [tc < 0.0 for tc in t]
    lm = _loo(av, jnp.minimum)
    x01, x23 = ng[0] != ng[1], ng[2] != ng[3]
    x45, x67 = ng[4] != ng[5], ng[6] != ng[7]
    tot = (x01 != x23) != (x45 != x67)
    msgs = []
    for c in range(W):
        mag = jnp.minimum(lm[c] * sval, 20.0)
        msgs.append(jnp.where(tot != ng[c], -mag, mag))
    return msgs


def _sc_body(x_hbm, a_hbm, out_hbm, vn, vn2, c2va, c2vb, avmem, sem):
    wid = lax.axis_index("s") * NC + lax.axis_index("c")

    pltpu.sync_copy(x_hbm.at[wid], vn)
    pltpu.sync_copy(a_hbm, avmem)

    zero = jnp.zeros((LANES,), jnp.float32)

    def zero_body(j, carry):
        sl = pl.ds(j * LANES, LANES)
        for g in range(W):
            for c in range(W):
                c2va[g, c, sl] = zero
                c2vb[g, c, sl] = zero
        return carry

    lax.fori_loop(0, VREGS, zero_body, 0)

    def one_phase(src, dst, c2v, row_of, sval):
        @plsc.parallel_loop(0, VREGS, unroll=2)
        def body_j(j):
            sl = pl.ds(j * LANES, LANES)
            for g in range(W):
                t = [src[row_of(g, c), sl] - c2v[g, c, sl] for c in range(W)]
                msgs = _minsum_group(t, sval)
                for c in range(W):
                    c2v[g, c, sl] = msgs[c]
                    dst[row_of(g, c), sl] = t[c] + msgs[c]

    def iter_body(it, carry):
        sval = 1.0 / (1.0 + jnp.exp(-avmem[it, :]))
        one_phase(vn, vn2, c2va, lambda g, c: W * g + c, sval)

        @pl.when(it > 0)
        def _():
            pltpu.make_async_copy(vn, out_hbm.at[it - 1, wid], sem).wait()

        one_phase(vn2, vn, c2vb, lambda g, c: W * c + g, sval)
        pltpu.async_copy(vn, out_hbm.at[it, wid], sem)
        return carry

    lax.fori_loop(0, ITERS, iter_body, 0)
    pltpu.make_async_copy(vn, out_hbm.at[ITERS - 1, wid], sem).wait()


@jax.jit
def _sc_decode(x3, a2d):
    mesh = plsc.VectorSubcoreMesh(core_axis_name="c", subcore_axis_name="s")
    run = functools.partial(
        pl.kernel,
        mesh=mesh,
        out_type=jax.ShapeDtypeStruct((ITERS, NW, N, BPW), jnp.float32),
        scratch_types=[
            pltpu.VMEM((N, BPW), jnp.float32),
            pltpu.VMEM((N, BPW), jnp.float32),
            pltpu.VMEM((W, W, BPW), jnp.float32),
            pltpu.VMEM((W, W, BPW), jnp.float32),
            pltpu.VMEM((ITERS, LANES), jnp.float32),
            pltpu.SemaphoreType.DMA,
        ],
    )(_sc_body)
    return run(x3, a2d)


def _tc_body(s_ref, x_ref, out_ref):
    vals = [x_ref[0, k] for k in range(N)]
    c2va = [jnp.zeros_like(vals[0]) for _ in range(N)]
    c2vb = [jnp.zeros_like(vals[0]) for _ in range(N)]

    for i in range(ITERS):
        sval = jax.nn.sigmoid(s_ref[0, i])
        for g in range(W):
            idx = [W * g + c for c in range(W)]
            t = [vals[idx[c]] - c2va[idx[c]] for c in range(W)]
            msgs = _minsum_group(t, sval)
            for c in range(W):
                c2va[idx[c]] = msgs[c]
                vals[idx[c]] = t[c] + msgs[c]
        for g in range(W):
            idx = [W * c + g for c in range(W)]
            t = [vals[idx[c]] - c2vb[idx[c]] for c in range(W)]
            msgs = _minsum_group(t, sval)
            for c in range(W):
                c2vb[idx[c]] = msgs[c]
                vals[idx[c]] = t[c] + msgs[c]
        for k in range(N):
            out_ref[0, i, k] = vals[k]


@jax.jit
def _tc_decode(x4, a_row):
    nb = x4.shape[0]
    return pl.pallas_call(
        _tc_body,
        grid=(nb,),
        in_specs=[
            pl.BlockSpec((1, ITERS), lambda g: (0, 0),
                         memory_space=pltpu.SMEM),
            pl.BlockSpec((1, N, 8, 128), lambda g: (g, 0, 0, 0)),
        ],
        out_specs=pl.BlockSpec((1, ITERS, N, 8, 128),
                               lambda g: (g, 0, 0, 0, 0)),
        out_shape=jax.ShapeDtypeStruct((nb, ITERS, N, 8, 128), jnp.float32),
    )(a_row, x4)


def kernel(channel_llr, cn_order, alphas, H_compact, mask):
    B, n = channel_llr.shape
    af = alphas.astype(jnp.float32)
    xt = channel_llr.T

    x3 = xt[:, :B_SC].reshape(n, NW, BPW).transpose(1, 0, 2)
    a2d = jnp.broadcast_to(af[:, None], (ITERS, LANES))
    out_sc = _sc_decode(x3, a2d)
    y_sc = out_sc.transpose(0, 1, 3, 2).reshape(ITERS, B_SC, n)

    nb = (B - B_SC) // TBLK
    x4 = xt[:, B_SC:].reshape(n, nb, 8, 128).transpose(1, 0, 2, 3)
    out_tc = _tc_decode(x4, af[None, :])
    y_tc = out_tc.transpose(1, 0, 3, 4, 2).reshape(ITERS, B - B_SC, n)

    return jnp.concatenate([y_sc, y_tc], axis=1)


# speedup vs baseline: 1.0003x; 1.0003x over previous
import functools

import jax
import jax.numpy as jnp
from jax import lax
from jax.experimental import pallas as pl
from jax.experimental.pallas import tpu as pltpu
from jax.experimental.pallas import tpu_sc as plsc

M, N, W, ITERS = 16, 64, 8, 10
NC, NS = 2, 16
NW = NC * NS
LANES = 16

B_SC = 1024
BPW = B_SC // NW
VREGS = BPW // LANES
TBLK = 1024


def _loo(vals, op):
    m01, m23 = op(vals[0], vals[1]), op(vals[2], vals[3])
    m45, m67 = op(vals[4], vals[5]), op(vals[6], vals[7])
    q03, q47 = op(m01, m23), op(m45, m67)
    h01, h23 = op(m23, q47), op(m01, q47)
    h45, h67 = op(q03, m67), op(q03, m45)
    return [
        op(vals[1], h01), op(vals[0], h01),
        op(vals[3], h23), op(vals[2], h23),
        op(vals[5], h45), op(vals[4], h45),
        op(vals[7], h67), op(vals[6], h67),
    ]


def _minsum_group(t, sval):
    av = [jnp.abs(tc) for tc in t]
    ng = [tc < 0.0 for tc in t]
    lm = _loo(av, jnp.minimum)
    x01, x23 = ng[0] != ng[1], ng[2] != ng[3]
    x45, x67 = ng[4] != ng[5], ng[6] != ng[7]
    tot = (x01 != x23) != (x45 != x67)
    msgs = []
    for c in range(W):
        mag = jnp.minimum(lm[c] * sval, 20.0)
        msgs.append(jnp.where(tot != ng[c], -mag, mag))
    return msgs


def _sc_body(x_hbm, a_hbm, out_hbm, vn, vn2, c2va, c2vb, avmem, sem):
    wid = lax.axis_index("s") * NC + lax.axis_index("c")

    pltpu.sync_copy(x_hbm.at[wid], vn)
    pltpu.sync_copy(a_hbm, avmem)

    zero = jnp.zeros((LANES,), jnp.float32)

    def zero_body(j, carry):
        sl = pl.ds(j * LANES, LANES)
        for g in range(W):
            for c in range(W):
                c2va[g, c, sl] = zero
                c2vb[g, c, sl] = zero
        return carry

    lax.fori_loop(0, VREGS, zero_body, 0)

    def one_phase(src, dst, c2v, row_of, sval):
        @plsc.parallel_loop(0, VREGS, unroll=2)
        def body_j(j):
            sl = pl.ds(j * LANES, LANES)
            for g in range(W):
                t = [src[row_of(g, c), sl] - c2v[g, c, sl] for c in range(W)]
                msgs = _minsum_group(t, sval)
                for c in range(W):
                    c2v[g, c, sl] = msgs[c]
                    dst[row_of(g, c), sl] = t[c] + msgs[c]

    def iter_body(it, carry):
        sval = 1.0 / (1.0 + jnp.exp(-avmem[it, :]))
        one_phase(vn, vn2, c2va, lambda g, c: W * g + c, sval)

        @pl.when(it > 0)
        def _():
            pltpu.make_async_copy(vn, out_hbm.at[it - 1, wid], sem).wait()

        one_phase(vn2, vn, c2vb, lambda g, c: W * c + g, sval)
        pltpu.async_copy(vn, out_hbm.at[it, wid], sem)
        return carry

    lax.fori_loop(0, ITERS, iter_body, 0)
    pltpu.make_async_copy(vn, out_hbm.at[ITERS - 1, wid], sem).wait()


@jax.jit
def _sc_decode(x3, a2d):
    mesh = plsc.VectorSubcoreMesh(core_axis_name="c", subcore_axis_name="s")
    run = functools.partial(
        pl.kernel,
        mesh=mesh,
        out_type=jax.ShapeDtypeStruct((ITERS, NW, N, BPW), jnp.float32),
        scratch_types=[
            pltpu.VMEM((N, BPW), jnp.float32),
            pltpu.VMEM((N, BPW), jnp.float32),
            pltpu.VMEM((W, W, BPW), jnp.float32),
            pltpu.VMEM((W, W, BPW), jnp.float32),
            pltpu.VMEM((ITERS, LANES), jnp.float32),
            pltpu.SemaphoreType.DMA,
        ],
    )(_sc_body)
    return run(x3, a2d)


def _tc_body(s_ref, x_ref, out_ref, vn_s, c2va_s, c2vb_s):
    zblk = jnp.zeros((8, 128), jnp.float32)
    for k in range(N):
        vn_s[k] = x_ref[0, k]
        c2va_s[k] = zblk
        c2vb_s[k] = zblk

    for i in range(ITERS):
        sval = jax.nn.sigmoid(s_ref[0, i])
        for g in range(W):
            idx = [W * g + c for c in range(W)]
            t = [vn_s[idx[c]] - c2va_s[idx[c]] for c in range(W)]
            msgs = _minsum_group(t, sval)
            for c in range(W):
                c2va_s[idx[c]] = msgs[c]
                vn_s[idx[c]] = t[c] + msgs[c]
        for g in range(W):
            idx = [W * c + g for c in range(W)]
            t = [vn_s[idx[c]] - c2vb_s[idx[c]] for c in range(W)]
            msgs = _minsum_group(t, sval)
            for c in range(W):
                c2vb_s[idx[c]] = msgs[c]
                v = t[c] + msgs[c]
                vn_s[idx[c]] = v
                out_ref[0, i, idx[c]] = v


@jax.jit
def _tc_decode(x4, a_row):
    nb = x4.shape[0]
    return pl.pallas_call(
        _tc_body,
        grid=(nb,),
        in_specs=[
            pl.BlockSpec((1, ITERS), lambda g: (0, 0),
                         memory_space=pltpu.SMEM),
            pl.BlockSpec((1, N, 8, 128), lambda g: (g, 0, 0, 0)),
        ],
        out_specs=pl.BlockSpec((1, ITERS, N, 8, 128),
                               lambda g: (g, 0, 0, 0, 0)),
        out_shape=jax.ShapeDtypeStruct((nb, ITERS, N, 8, 128), jnp.float32),
        scratch_shapes=[
            pltpu.VMEM((N, 8, 128), jnp.float32),
            pltpu.VMEM((N, 8, 128), jnp.float32),
            pltpu.VMEM((N, 8, 128), jnp.float32),
        ],
    )(a_row, x4)


def kernel(channel_llr, cn_order, alphas, H_compact, mask):
    B, n = channel_llr.shape
    af = alphas.astype(jnp.float32)
    xt = channel_llr.T

    x3 = xt[:, :B_SC].reshape(n, NW, BPW).transpose(1, 0, 2)
    a2d = jnp.broadcast_to(af[:, None], (ITERS, LANES))
    out_sc = _sc_decode(x3, a2d)
    y_sc = out_sc.transpose(0, 1, 3, 2).reshape(ITERS, B_SC, n)

    nb = (B - B_SC) // TBLK
    x4 = xt[:, B_SC:].reshape(n, nb, 8, 128).transpose(1, 0, 2, 3)
    out_tc = _tc_decode(x4, af[None, :])
    y_tc = out_tc.transpose(1, 0, 3, 4, 2).reshape(ITERS, B - B_SC, n)

    return jnp.concatenate([y_sc, y_tc], axis=1)
